# baseline (speedup 1.0000x reference)
import jax
import jax.numpy as jnp
from jax.experimental import pallas as pl
from jax.experimental.pallas import tpu as pltpu

_DN = (((1,), (0,)), ((), ()))


def _sage_kernel(adj_ref, x_ref, w_ref, out_ref, y2_ref, *, bm):
    i = pl.program_id(0)
    f_in = x_ref.shape[1]

    @pl.when(i == 0)
    def _build_y2():
        w2 = w_ref[pl.ds(f_in, f_in), :]
        y2_ref[...] = jax.lax.dot_general(
            x_ref[...], w2, _DN, preferred_element_type=jnp.float32,
            precision=jax.lax.Precision.DEFAULT)

    w1 = w_ref[pl.ds(0, f_in), :]
    x_band = x_ref[pl.ds(i * bm, bm), :]
    self_term = jax.lax.dot_general(
        x_band, w1, _DN, preferred_element_type=jnp.float32,
        precision=jax.lax.Precision.DEFAULT)
    out_ref[...] = self_term + jax.lax.dot_general(
        adj_ref[...], y2_ref[...], _DN, preferred_element_type=jnp.float32,
        precision=jax.lax.Precision.DEFAULT)


def kernel(input, adj, weight):
    n, f_in = input.shape
    f_out = weight.shape[1]
    bm = min(n, 200)

    import functools
    body = functools.partial(_sage_kernel, bm=bm)
    out = pl.pallas_call(
        body,
        grid=(n // bm,),
        in_specs=[
            pl.BlockSpec((bm, n), lambda i: (i, 0)),
            pl.BlockSpec((n, f_in), lambda i: (0, 0)),
            pl.BlockSpec((2 * f_in, f_out), lambda i: (0, 0)),
        ],
        out_specs=pl.BlockSpec((bm, f_out), lambda i: (i, 0)),
        out_shape=jax.ShapeDtypeStruct((n, f_out), jnp.float32),
        scratch_shapes=[pltpu.VMEM((n, f_out), jnp.float32)],
        compiler_params=pltpu.CompilerParams(
            dimension_semantics=("arbitrary",)),
    )(adj, input, weight)
    return out


# speedup vs baseline: 1.0185x; 1.0185x over previous
import functools

import jax
import jax.numpy as jnp
from jax.experimental import pallas as pl
from jax.experimental.pallas import tpu as pltpu

_DN = (((1,), (0,)), ((), ()))
_PREC = jax.lax.Precision.DEFAULT


def _sage_kernel(adj_a_ref, adj_b_ref, x_ref, w_ref,
                 out_ref, y2_ref, *, bm, half):
    i = pl.program_id(0)
    f_in = x_ref.shape[1]

    @pl.when(i == 0)
    def _build_y2():
        w2 = w_ref[pl.ds(f_in, f_in), :]
        y2_ref[...] = jax.lax.dot_general(
            x_ref[...], w2, _DN, preferred_element_type=jnp.float32,
            precision=_PREC)

    w1 = w_ref[pl.ds(0, f_in), :]

    def band(adj_ref, row0):
        x_band = x_ref[pl.ds(row0, bm), :]
        self_term = jax.lax.dot_general(
            x_band, w1, _DN, preferred_element_type=jnp.float32,
            precision=_PREC)
        return self_term + jax.lax.dot_general(
            adj_ref[0], y2_ref[...], _DN,
            preferred_element_type=jnp.float32, precision=_PREC)

    out_ref[0] = band(adj_a_ref, i * bm)
    out_ref[1] = band(adj_b_ref, half + i * bm)


def kernel(input, adj, weight):
    n, f_in = input.shape
    f_out = weight.shape[1]
    bm = min(n // 2, 200)
    half = n // 2
    steps = half // bm
    adj3 = adj.reshape(2, half, n)

    body = functools.partial(_sage_kernel, bm=bm, half=half)
    out = pl.pallas_call(
        body,
        grid=(steps,),
        in_specs=[
            pl.BlockSpec((1, bm, n), lambda i: (0, i, 0)),
            pl.BlockSpec((1, bm, n), lambda i: (1, i, 0)),
            pl.BlockSpec((n, f_in), lambda i: (0, 0)),
            pl.BlockSpec((2 * f_in, f_out), lambda i: (0, 0)),
        ],
        out_specs=pl.BlockSpec((2, bm, f_out), lambda i: (0, i, 0)),
        out_shape=jax.ShapeDtypeStruct((2, half, f_out), jnp.float32),
        scratch_shapes=[pltpu.VMEM((n, f_out), jnp.float32)],
        compiler_params=pltpu.CompilerParams(
            dimension_semantics=("arbitrary",)),
    )(adj3, adj3, input, weight)
    return out.reshape(n, f_out)


# speedup vs baseline: 1.0201x; 1.0015x over previous
import functools

import jax
import jax.numpy as jnp
from jax.experimental import pallas as pl
from jax.experimental.pallas import tpu as pltpu

_DN = (((1,), (0,)), ((), ()))
_PREC = jax.lax.Precision.DEFAULT


def _sage_kernel(adj_ref, x_ref, w_ref, out_ref, y2_ref, *, bm):
    i = pl.program_id(0)
    f_in = x_ref.shape[1]

    @pl.when(i == 0)
    def _build_y2():
        w2 = w_ref[pl.ds(f_in, f_in), :]
        y2_ref[...] = jax.lax.dot_general(
            x_ref[...], w2, _DN, preferred_element_type=jnp.float32,
            precision=_PREC)

    w1 = w_ref[pl.ds(0, f_in), :]
    x_band = x_ref[pl.ds(i * bm, bm), :]
    self_term = jax.lax.dot_general(
        x_band, w1, _DN, preferred_element_type=jnp.float32,
        precision=_PREC)
    out_ref[...] = self_term + jax.lax.dot_general(
        adj_ref[...], y2_ref[...], _DN,
        preferred_element_type=jnp.float32, precision=_PREC)


def kernel(input, adj, weight):
    n, f_in = input.shape
    f_out = weight.shape[1]
    bm = min(n, 400)

    body = functools.partial(_sage_kernel, bm=bm)
    out = pl.pallas_call(
        body,
        grid=(n // bm,),
        in_specs=[
            pl.BlockSpec((bm, n), lambda i: (i, 0)),
            pl.BlockSpec((n, f_in), lambda i: (0, 0)),
            pl.BlockSpec((2 * f_in, f_out), lambda i: (0, 0)),
        ],
        out_specs=pl.BlockSpec((bm, f_out), lambda i: (i, 0)),
        out_shape=jax.ShapeDtypeStruct((n, f_out), jnp.float32),
        scratch_shapes=[pltpu.VMEM((n, f_out), jnp.float32)],
        compiler_params=pltpu.CompilerParams(
            dimension_semantics=("arbitrary",)),
    )(adj, input, weight)
    return out
